# Initial kernel scaffold; baseline (speedup 1.0000x reference)
import jax
import jax.numpy as jnp
from jax.experimental import pallas as pl


def kernel(x, edge_index, W1l, W1r, att1, b1, W2l, W2r, att2, b2):
    raise NotImplementedError("write your pallas kernel here")


# speedup vs baseline: 1.0362x; 1.0362x over previous
import functools

import jax
import jax.numpy as jnp
from jax.experimental import pallas as pl
from jax.experimental.pallas import tpu as pltpu

N = 10000
E = 320000
F = 128
H1 = 8
C1 = 64
C2 = 40


def _mm_kernel(x_ref, wl_ref, wr_ref, ol_ref, or_ref):
    x = x_ref[...]
    ol_ref[...] = jnp.dot(x, wl_ref[...], preferred_element_type=jnp.float32)
    or_ref[...] = jnp.dot(x, wr_ref[...], preferred_element_type=jnp.float32)


def _dual_matmul(x, wl, wr, block_rows=1000):
    n, f = x.shape
    k = wl.shape[1]
    grid = (n // block_rows,)
    return pl.pallas_call(
        _mm_kernel,
        grid=grid,
        in_specs=[
            pl.BlockSpec((block_rows, f), lambda i: (i, 0)),
            pl.BlockSpec((f, k), lambda i: (0, 0)),
            pl.BlockSpec((f, k), lambda i: (0, 0)),
        ],
        out_specs=[
            pl.BlockSpec((block_rows, k), lambda i: (i, 0)),
            pl.BlockSpec((block_rows, k), lambda i: (i, 0)),
        ],
        out_shape=[
            jax.ShapeDtypeStruct((n, k), jnp.float32),
            jax.ShapeDtypeStruct((n, k), jnp.float32),
        ],
    )(x, wl, wr)


def _edge_phase(xl, xr, src, dst, att, heads, out_ch, num_nodes):
    xj = xl[src].reshape(-1, heads, out_ch)
    xi = xr[dst].reshape(-1, heads, out_ch)
    s = jax.nn.leaky_relu(xi + xj, negative_slope=0.2)
    alpha = jnp.sum(s * att[None, :, :], axis=-1)
    amax = jax.ops.segment_max(alpha, dst, num_segments=num_nodes)
    amax = jnp.where(jnp.isfinite(amax), amax, 0.0)
    ex = jnp.exp(alpha - amax[dst])
    denom = jax.ops.segment_sum(ex, dst, num_segments=num_nodes)
    alpha_n = ex / (denom[dst] + 1e-16)
    msg = xj * alpha_n[:, :, None]
    out = jax.ops.segment_sum(msg, dst, num_segments=num_nodes)
    return out.reshape(num_nodes, heads * out_ch)


def kernel(x, edge_index, W1l, W1r, att1, b1, W2l, W2r, att2, b2):
    loop = jnp.arange(N, dtype=edge_index.dtype)
    src = jnp.concatenate([edge_index[0], loop])
    dst = jnp.concatenate([edge_index[1], loop])

    xl1, xr1 = _dual_matmul(x, W1l, W1r)
    h = _edge_phase(xl1, xr1, src, dst, att1, H1, C1, N) + b1
    h = jax.nn.relu(h)
    hl2, hr2 = _dual_matmul(h, W2l, W2r)
    out = _edge_phase(hl2, hr2, src, dst, att2, 1, C2, N) + b2
    return out
